# baseline (speedup 1.0000x reference)
import functools

import jax
import jax.numpy as jnp
from jax import lax
from jax.experimental import pallas as pl
from jax.experimental.pallas import tpu as pltpu
from jax.experimental.pallas import tpu_sc as plsc

NE = 8
H = 1024
F = 2048
T = 2048
BLK = 256
NBE = 8
NBU = 24
TRASH = NBU
NPAD = (NBU + 1) * BLK
NW = 32
TPW = T // NW


def _router_body(x_ref, gw_ref, pos_ref, wpair_ref, meta_ref):
    x = x_ref[...]
    logits = lax.dot_general(gw_ref[...], x, (((1,), (1,)), ((), ())),
                             preferred_element_type=jnp.float32)
    m = jnp.max(logits, axis=0, keepdims=True)
    ex = jnp.exp(logits - m)
    p = ex / jnp.sum(ex, axis=0, keepdims=True)

    ii = lax.broadcasted_iota(jnp.int32, (NE, T), 0)
    big = jnp.int32(NE)
    pm1 = jnp.max(p, axis=0, keepdims=True)
    idx1 = jnp.min(jnp.where(p == pm1, ii, big), axis=0, keepdims=True)
    mask1 = ii == idx1
    p2 = jnp.where(mask1, -1.0, p)
    pm2 = jnp.max(p2, axis=0, keepdims=True)
    idx2 = jnp.min(jnp.where(p2 == pm2, ii, big), axis=0, keepdims=True)

    wsum = pm1 + pm2
    wpair_ref[...] = jnp.concatenate([pm1, pm2], axis=0) / wsum

    keys = jnp.concatenate([idx1, idx2], axis=0)
    keys32 = keys.reshape(32, 128)

    iu = lax.broadcasted_iota(jnp.int32, (128, 128), 0)
    ju = lax.broadcasted_iota(jnp.int32, (128, 128), 1)
    U = (iu <= ju).astype(jnp.float32)
    il = lax.broadcasted_iota(jnp.int32, (32, 32), 0)
    jl = lax.broadcasted_iota(jnp.int32, (32, 32), 1)
    L = (jl < il).astype(jnp.float32)

    pos = jnp.zeros((32, 128), jnp.float32)
    runblk = jnp.float32(0.0)
    lanes = lax.broadcasted_iota(jnp.int32, (1, 64), 1)
    meta = jnp.zeros((1, 64), jnp.float32)
    for e in range(NE):
        me = (keys32 == e).astype(jnp.float32)
        incl = lax.dot_general(me, U, (((1,), (0,)), ((), ())),
                               preferred_element_type=jnp.float32)
        rowtot = incl[:, 127:128]
        rowoff = lax.dot_general(L, rowtot, (((1,), (0,)), ((), ())),
                                 preferred_element_type=jnp.float32)
        rank = incl - me + rowoff
        cnt = jnp.sum(me)
        nb_e = jnp.floor((cnt + float(BLK - 1)) * (1.0 / BLK))
        pos = pos + me * (float(BLK) * runblk + rank)
        meta = meta + jnp.where(lanes == e, nb_e, 0.0)
        meta = meta + jnp.where(lanes == NE + e, runblk, 0.0)
        runblk = runblk + nb_e

    pos_ref[...] = pos.astype(jnp.int32)
    meta_ref[...] = meta.astype(jnp.int32)


def _run_router(x_flat, gate_w):
    return pl.pallas_call(
        _router_body,
        out_shape=(
            jax.ShapeDtypeStruct((32, 128), jnp.int32),
            jax.ShapeDtypeStruct((2, T), jnp.float32),
            jax.ShapeDtypeStruct((1, 64), jnp.int32),
        ),
    )(x_flat, gate_w)


@functools.lru_cache(maxsize=None)
def _make_scatter_sc():
    @functools.partial(
        pl.kernel,
        out_type=jax.ShapeDtypeStruct((NPAD, H), jnp.float32),
        mesh=plsc.VectorSubcoreMesh(core_axis_name="c", subcore_axis_name="s"),
        scratch_types=[
            pltpu.VMEM((TPW, H), jnp.float32),
            pltpu.VMEM((2, TPW), jnp.int32),
            pltpu.SemaphoreType.DMA((2,)),
            pltpu.SemaphoreType.DMA((2,)),
        ],
    )
    def _scatter_sc(x_hbm, islot_hbm, xs_hbm, xv, idxv, lsem, ssem):
        wid = lax.axis_index("s") * 2 + lax.axis_index("c")
        base = wid * TPW
        CH = TPW // 2
        pltpu.sync_copy(islot_hbm.at[wid], idxv)
        loads = []
        for c in range(2):
            loads.append(pltpu.async_copy(
                x_hbm.at[pl.ds(base + c * CH, CH)],
                xv.at[pl.ds(c * CH, CH)], lsem.at[c]))
        pend = []
        for c in range(2):
            loads[c].wait()
            pend.append(pltpu.async_copy(
                xv.at[pl.ds(c * CH, CH)],
                xs_hbm.at[idxv.at[0, pl.ds(c * CH, CH)]], ssem.at[c]))
            pend.append(pltpu.async_copy(
                xv.at[pl.ds(c * CH, CH)],
                xs_hbm.at[idxv.at[1, pl.ds(c * CH, CH)]], ssem.at[c]))
        for cp in pend:
            cp.wait()

    return _scatter_sc


def _expert_body(nb_ref, bs_ref, xs_ref, gpw_hbm, upw_hbm, dpw_hbm, ys_ref,
                 gbuf, ubuf, dbuf, gsem, usem, dsem):
    e = pl.program_id(0)
    b = pl.program_id(1)

    def fetch(ee, p):
        pltpu.make_async_copy(gpw_hbm.at[ee], gbuf.at[p], gsem.at[p]).start()
        pltpu.make_async_copy(upw_hbm.at[ee], ubuf.at[p], usem.at[p]).start()
        pltpu.make_async_copy(dpw_hbm.at[ee], dbuf.at[p], dsem.at[p]).start()

    @pl.when(jnp.logical_and(e == 0, b == 0))
    def _():
        fetch(0, 0)

    @pl.when(b == 0)
    def _():
        p = e % 2
        pltpu.make_async_copy(gpw_hbm.at[e], gbuf.at[p], gsem.at[p]).wait()

    @pl.when(b < nb_ref[e])
    def _():
        p = e % 2
        xb = xs_ref[...]
        g = lax.dot_general(xb, gbuf[p], (((1,), (1,)), ((), ())),
                            preferred_element_type=jnp.float32)

        @pl.when(b == 0)
        def _():
            pltpu.make_async_copy(upw_hbm.at[e], ubuf.at[p], usem.at[p]).wait()

        u = lax.dot_general(xb, ubuf[p], (((1,), (1,)), ((), ())),
                            preferred_element_type=jnp.float32)
        h = g * (1.0 / (1.0 + jnp.exp(-g))) * u

        @pl.when(b == 0)
        def _():
            pltpu.make_async_copy(dpw_hbm.at[e], dbuf.at[p], dsem.at[p]).wait()

        ys_ref[...] = lax.dot_general(h, dbuf[p], (((1,), (1,)), ((), ())),
                                      preferred_element_type=jnp.float32)

    @pl.when(jnp.logical_and(b == 0, nb_ref[e] == 0))
    def _():
        p = e % 2
        pltpu.make_async_copy(upw_hbm.at[e], ubuf.at[p], usem.at[p]).wait()
        pltpu.make_async_copy(dpw_hbm.at[e], dbuf.at[p], dsem.at[p]).wait()

    @pl.when(jnp.logical_and(b == 0, e < NE - 1))
    def _():
        fetch(e + 1, (e + 1) % 2)


def _data_block(e, b, nb, bs):
    return jnp.where(b < nb[e], bs[e] + b, TRASH)


def _run_experts(nb, bs, xs, gpw, upw, dpw):
    return pl.pallas_call(
        _expert_body,
        grid_spec=pltpu.PrefetchScalarGridSpec(
            num_scalar_prefetch=2,
            grid=(NE, NBE),
            in_specs=[
                pl.BlockSpec((BLK, H), lambda e, b, nb, bs: (_data_block(e, b, nb, bs), 0)),
                pl.BlockSpec(memory_space=pl.ANY),
                pl.BlockSpec(memory_space=pl.ANY),
                pl.BlockSpec(memory_space=pl.ANY),
            ],
            out_specs=pl.BlockSpec((BLK, H), lambda e, b, nb, bs: (_data_block(e, b, nb, bs), 0)),
            scratch_shapes=[
                pltpu.VMEM((2, F, H), jnp.float32),
                pltpu.VMEM((2, F, H), jnp.float32),
                pltpu.VMEM((2, H, F), jnp.float32),
                pltpu.SemaphoreType.DMA((2,)),
                pltpu.SemaphoreType.DMA((2,)),
                pltpu.SemaphoreType.DMA((2,)),
            ],
        ),
        out_shape=jax.ShapeDtypeStruct((NPAD, H), jnp.float32),
        compiler_params=pltpu.CompilerParams(
            dimension_semantics=("arbitrary", "arbitrary"),
        ),
    )(nb, bs, xs, gpw, upw, dpw)


@functools.lru_cache(maxsize=None)
def _make_combine_sc():
    @functools.partial(
        pl.kernel,
        out_type=jax.ShapeDtypeStruct((T, H), jnp.float32),
        mesh=plsc.VectorSubcoreMesh(core_axis_name="c", subcore_axis_name="s"),
        scratch_types=[
            pltpu.VMEM((2, TPW // 4, H), jnp.float32),
            pltpu.VMEM((2, TPW // 4, H), jnp.float32),
            pltpu.VMEM((TPW // 4, H), jnp.float32),
            pltpu.VMEM((2, TPW), jnp.int32),
            pltpu.VMEM((TPW, 16), jnp.float32),
            pltpu.VMEM((TPW, 16), jnp.float32),
            pltpu.SemaphoreType.DMA((2,)),
        ],
    )
    def _combine_sc(ys_hbm, islot_hbm, wb0_hbm, wb1_hbm, out_hbm,
                    r0, r1, ob, idxv, wv0, wv1, sem):
        wid = lax.axis_index("s") * 2 + lax.axis_index("c")
        base = wid * TPW
        pltpu.sync_copy(islot_hbm.at[wid], idxv)
        pltpu.sync_copy(wb0_hbm.at[pl.ds(base, TPW)], wv0)
        pltpu.sync_copy(wb1_hbm.at[pl.ds(base, TPW)], wv1)
        CH = TPW // 4
        NC = 4

        def gathers(c):
            p = c % 2
            c0 = pltpu.async_copy(
                ys_hbm.at[idxv.at[0, pl.ds(c * CH, CH)]], r0.at[p], sem.at[p])
            c1 = pltpu.async_copy(
                ys_hbm.at[idxv.at[1, pl.ds(c * CH, CH)]], r1.at[p], sem.at[p])
            return c0, c1

        pend = gathers(0)
        for c in range(NC):
            p = c % 2
            cur = pend
            if c < NC - 1:
                pend = gathers(c + 1)
            cur[0].wait()
            cur[1].wait()

            def body(i, _):
                t = c * CH + i
                w0 = wv0[t, :]
                w1 = wv1[t, :]
                for cix in range(H // 16):
                    sl = pl.ds(cix * 16, 16)
                    ob[i, sl] = r0[p, i, sl] * w0 + r1[p, i, sl] * w1
                return 0

            lax.fori_loop(0, CH, body, 0)
            pltpu.sync_copy(ob, out_hbm.at[pl.ds(base + c * CH, CH)])

    return _combine_sc


@jax.jit
def kernel(x, gate_w, gate_proj_w, up_proj_w, down_proj_w):
    Bs, Ss, Hh = x.shape
    x_flat = x.reshape(T, H)

    pos32, wpair, meta2d = _run_router(x_flat, gate_w)

    posf = pos32.reshape(2, NW, TPW)
    islot01 = posf.transpose(1, 0, 2)
    wb0 = jnp.broadcast_to(wpair[0][:, None], (T, 16))
    wb1 = jnp.broadcast_to(wpair[1][:, None], (T, 16))
    meta = meta2d.reshape(64)
    nb = meta[:NE]
    bs = meta[NE:2 * NE]

    xs = _make_scatter_sc()(x_flat, islot01)
    ys = _run_experts(nb, bs, xs, gate_proj_w, up_proj_w, down_proj_w)
    out = _make_combine_sc()(ys, islot01, wb0, wb1)

    return out.reshape(Bs, Ss, Hh)


# speedup vs baseline: 1.1870x; 1.1870x over previous
import functools

import jax
import jax.numpy as jnp
from jax import lax
from jax.experimental import pallas as pl
from jax.experimental.pallas import tpu as pltpu
from jax.experimental.pallas import tpu_sc as plsc

NE = 8
H = 1024
F = 2048
T = 2048
BLK = 256
NBE = 8
NBU = 24
TRASH = NBU
NPAD = (NBU + 1) * BLK
NW = 32
TPW = T // NW


def _router_body(x_ref, gw_ref, pos_ref, wpair_ref, meta_ref):
    x = x_ref[...]
    logits = lax.dot_general(gw_ref[...], x, (((1,), (1,)), ((), ())),
                             preferred_element_type=jnp.float32)
    m = jnp.max(logits, axis=0, keepdims=True)
    ex = jnp.exp(logits - m)
    p = ex / jnp.sum(ex, axis=0, keepdims=True)

    ii = lax.broadcasted_iota(jnp.int32, (NE, T), 0)
    big = jnp.int32(NE)
    pm1 = jnp.max(p, axis=0, keepdims=True)
    idx1 = jnp.min(jnp.where(p == pm1, ii, big), axis=0, keepdims=True)
    mask1 = ii == idx1
    p2 = jnp.where(mask1, -1.0, p)
    pm2 = jnp.max(p2, axis=0, keepdims=True)
    idx2 = jnp.min(jnp.where(p2 == pm2, ii, big), axis=0, keepdims=True)

    wsum = pm1 + pm2
    wpair_ref[...] = jnp.concatenate([pm1, pm2], axis=0) / wsum

    keys = jnp.concatenate([idx1, idx2], axis=0)
    keys32 = keys.reshape(32, 128)

    iu = lax.broadcasted_iota(jnp.int32, (128, 128), 0)
    ju = lax.broadcasted_iota(jnp.int32, (128, 128), 1)
    U = (iu <= ju).astype(jnp.float32)
    il = lax.broadcasted_iota(jnp.int32, (32, 32), 0)
    jl = lax.broadcasted_iota(jnp.int32, (32, 32), 1)
    L = (jl < il).astype(jnp.float32)

    pos = jnp.zeros((32, 128), jnp.float32)
    runblk = jnp.float32(0.0)
    lanes = lax.broadcasted_iota(jnp.int32, (1, 64), 1)
    meta = jnp.zeros((1, 64), jnp.float32)
    for e in range(NE):
        me = (keys32 == e).astype(jnp.float32)
        incl = lax.dot_general(me, U, (((1,), (0,)), ((), ())),
                               preferred_element_type=jnp.float32)
        rowtot = incl[:, 127:128]
        rowoff = lax.dot_general(L, rowtot, (((1,), (0,)), ((), ())),
                                 preferred_element_type=jnp.float32)
        rank = incl - me + rowoff
        cnt = jnp.sum(me)
        nb_e = jnp.floor((cnt + float(BLK - 1)) * (1.0 / BLK))
        pos = pos + me * (float(BLK) * runblk + rank)
        meta = meta + jnp.where(lanes == e, nb_e, 0.0)
        meta = meta + jnp.where(lanes == NE + e, runblk, 0.0)
        runblk = runblk + nb_e

    pos_ref[...] = pos.astype(jnp.int32)
    meta_ref[...] = meta.astype(jnp.int32)


def _run_router(x_flat, gate_w):
    return pl.pallas_call(
        _router_body,
        out_shape=(
            jax.ShapeDtypeStruct((32, 128), jnp.int32),
            jax.ShapeDtypeStruct((2, T), jnp.float32),
            jax.ShapeDtypeStruct((1, 64), jnp.int32),
        ),
    )(x_flat, gate_w)


@functools.lru_cache(maxsize=None)
def _make_scatter_sc():
    @functools.partial(
        pl.kernel,
        out_type=jax.ShapeDtypeStruct((NPAD, H), jnp.float32),
        mesh=plsc.VectorSubcoreMesh(core_axis_name="c", subcore_axis_name="s"),
        scratch_types=[
            pltpu.VMEM((TPW, H), jnp.float32),
            pltpu.VMEM((2, TPW), jnp.int32),
            pltpu.SemaphoreType.DMA((2,)),
            pltpu.SemaphoreType.DMA((2,)),
        ],
    )
    def _scatter_sc(x_hbm, islot_hbm, xs_hbm, xv, idxv, lsem, ssem):
        wid = lax.axis_index("s") * 2 + lax.axis_index("c")
        base = wid * TPW
        CH = TPW // 2
        pltpu.sync_copy(islot_hbm.at[wid], idxv)
        loads = []
        for c in range(2):
            loads.append(pltpu.async_copy(
                x_hbm.at[pl.ds(base + c * CH, CH)],
                xv.at[pl.ds(c * CH, CH)], lsem.at[c]))
        pend = []
        for c in range(2):
            loads[c].wait()
            pend.append(pltpu.async_copy(
                xv.at[pl.ds(c * CH, CH)],
                xs_hbm.at[idxv.at[0, pl.ds(c * CH, CH)]], ssem.at[c]))
            pend.append(pltpu.async_copy(
                xv.at[pl.ds(c * CH, CH)],
                xs_hbm.at[idxv.at[1, pl.ds(c * CH, CH)]], ssem.at[c]))
        for cp in pend:
            cp.wait()

    return _scatter_sc


def _expert_body(nb_ref, bs_ref, xs_ref, gpw_hbm, upw_hbm, dpw_hbm, ys_ref,
                 gbuf, ubuf, dbuf, gsem, usem, dsem):
    e = pl.program_id(0)
    b = pl.program_id(1)

    def fetch(ee, p):
        pltpu.make_async_copy(gpw_hbm.at[ee], gbuf.at[p], gsem.at[p]).start()
        pltpu.make_async_copy(upw_hbm.at[ee], ubuf.at[p], usem.at[p]).start()
        pltpu.make_async_copy(dpw_hbm.at[ee], dbuf.at[p], dsem.at[p]).start()

    @pl.when(jnp.logical_and(e == 0, b == 0))
    def _():
        fetch(0, 0)

    @pl.when(b == 0)
    def _():
        p = e % 2
        pltpu.make_async_copy(gpw_hbm.at[e], gbuf.at[p], gsem.at[p]).wait()
        pltpu.make_async_copy(upw_hbm.at[e], ubuf.at[p], usem.at[p]).wait()
        pltpu.make_async_copy(dpw_hbm.at[e], dbuf.at[p], dsem.at[p]).wait()

        @pl.when(e < NE - 1)
        def _():
            fetch(e + 1, (e + 1) % 2)

    @pl.when(b < nb_ref[e])
    def _():
        p = e % 2
        xb = xs_ref[...]
        g = lax.dot_general(xb, gbuf[p], (((1,), (1,)), ((), ())),
                            preferred_element_type=jnp.float32)
        u = lax.dot_general(xb, ubuf[p], (((1,), (1,)), ((), ())),
                            preferred_element_type=jnp.float32)
        h = g * (1.0 / (1.0 + jnp.exp(-g))) * u
        ys_ref[...] = lax.dot_general(h, dbuf[p], (((1,), (1,)), ((), ())),
                                      preferred_element_type=jnp.float32)


def _data_block(e, b, nb, bs):
    return jnp.where(b < nb[e], bs[e] + b, TRASH)


def _run_experts(nb, bs, xs, gpw, upw, dpw):
    return pl.pallas_call(
        _expert_body,
        grid_spec=pltpu.PrefetchScalarGridSpec(
            num_scalar_prefetch=2,
            grid=(NE, NBE),
            in_specs=[
                pl.BlockSpec((BLK, H), lambda e, b, nb, bs: (_data_block(e, b, nb, bs), 0)),
                pl.BlockSpec(memory_space=pl.ANY),
                pl.BlockSpec(memory_space=pl.ANY),
                pl.BlockSpec(memory_space=pl.ANY),
            ],
            out_specs=pl.BlockSpec((BLK, H), lambda e, b, nb, bs: (_data_block(e, b, nb, bs), 0)),
            scratch_shapes=[
                pltpu.VMEM((2, F, H), jnp.float32),
                pltpu.VMEM((2, F, H), jnp.float32),
                pltpu.VMEM((2, H, F), jnp.float32),
                pltpu.SemaphoreType.DMA((2,)),
                pltpu.SemaphoreType.DMA((2,)),
                pltpu.SemaphoreType.DMA((2,)),
            ],
        ),
        out_shape=jax.ShapeDtypeStruct((NPAD, H), jnp.float32),
        compiler_params=pltpu.CompilerParams(
            dimension_semantics=("arbitrary", "arbitrary"),
        ),
    )(nb, bs, xs, gpw, upw, dpw)


@functools.lru_cache(maxsize=None)
def _make_combine_sc():
    @functools.partial(
        pl.kernel,
        out_type=jax.ShapeDtypeStruct((T, H), jnp.float32),
        mesh=plsc.VectorSubcoreMesh(core_axis_name="c", subcore_axis_name="s"),
        scratch_types=[
            pltpu.VMEM((2, TPW // 4, H), jnp.float32),
            pltpu.VMEM((2, TPW // 4, H), jnp.float32),
            pltpu.VMEM((TPW // 4, H), jnp.float32),
            pltpu.VMEM((2, TPW), jnp.int32),
            pltpu.VMEM((TPW, 16), jnp.float32),
            pltpu.VMEM((TPW, 16), jnp.float32),
            pltpu.SemaphoreType.DMA((2,)),
        ],
    )
    def _combine_sc(ys_hbm, islot_hbm, wb0_hbm, wb1_hbm, out_hbm,
                    r0, r1, ob, idxv, wv0, wv1, sem):
        wid = lax.axis_index("s") * 2 + lax.axis_index("c")
        base = wid * TPW
        pltpu.sync_copy(islot_hbm.at[wid], idxv)
        pltpu.sync_copy(wb0_hbm.at[pl.ds(base, TPW)], wv0)
        pltpu.sync_copy(wb1_hbm.at[pl.ds(base, TPW)], wv1)
        CH = TPW // 4
        NC = 4

        def gathers(c):
            p = c % 2
            c0 = pltpu.async_copy(
                ys_hbm.at[idxv.at[0, pl.ds(c * CH, CH)]], r0.at[p], sem.at[p])
            c1 = pltpu.async_copy(
                ys_hbm.at[idxv.at[1, pl.ds(c * CH, CH)]], r1.at[p], sem.at[p])
            return c0, c1

        pend = gathers(0)
        for c in range(NC):
            p = c % 2
            cur = pend
            if c < NC - 1:
                pend = gathers(c + 1)
            cur[0].wait()
            cur[1].wait()

            def body(i, _):
                t = c * CH + i
                w0 = wv0[t, :]
                w1 = wv1[t, :]
                for cix in range(H // 16):
                    sl = pl.ds(cix * 16, 16)
                    ob[i, sl] = r0[p, i, sl] * w0 + r1[p, i, sl] * w1
                return 0

            lax.fori_loop(0, CH, body, 0)
            pltpu.sync_copy(ob, out_hbm.at[pl.ds(base + c * CH, CH)])

    return _combine_sc


@jax.jit
def kernel(x, gate_w, gate_proj_w, up_proj_w, down_proj_w):
    Bs, Ss, Hh = x.shape
    x_flat = x.reshape(T, H)

    pos32, wpair, meta2d = _run_router(x_flat, gate_w)

    posf = pos32.reshape(2, NW, TPW)
    islot01 = posf.transpose(1, 0, 2)
    wb0 = jnp.broadcast_to(wpair[0][:, None], (T, 16))
    wb1 = jnp.broadcast_to(wpair[1][:, None], (T, 16))
    meta = meta2d.reshape(64)
    nb = meta[:NE]
    bs = meta[NE:2 * NE]

    xs = _make_scatter_sc()(x_flat, islot01)
    ys = _run_experts(nb, bs, xs, gate_proj_w, up_proj_w, down_proj_w)
    out = _make_combine_sc()(ys, islot01, wb0, wb1)

    return out.reshape(Bs, Ss, Hh)
